# Initial kernel scaffold; baseline (speedup 1.0000x reference)
import jax
import jax.numpy as jnp
from jax.experimental import pallas as pl


def kernel(fea, res, queue, batch_size):
    raise NotImplementedError("write your pallas kernel here")


# speedup vs baseline: 3.0974x; 3.0974x over previous
import jax
import jax.numpy as jnp
from jax.experimental import pallas as pl
from jax.experimental.pallas import tpu as pltpu

NCLASS = 7
INNER = 256
QUEUE_LEN = 8827
CHUNK = 2048
QCHUNK = 1024


def _keys_body(res_ref, fea_ref, out_ref, acc_ref, cnt_ref, *, nb, nch):
    b = pl.program_id(0)
    j = pl.program_id(1)

    @pl.when((b == 0) & (j == 0))
    def _init():
        acc_ref[...] = jnp.zeros_like(acc_ref)
        cnt_ref[...] = jnp.zeros_like(cnt_ref)

    r = res_ref[0]
    best = r[0:1]
    idx = jnp.zeros((1, CHUNK), jnp.int32)
    for c in range(1, NCLASS):
        row = r[c:c + 1]
        gt = row > best
        best = jnp.where(gt, row, best)
        idx = jnp.where(gt, c, idx)

    classes = jax.lax.broadcasted_iota(jnp.int32, (8, CHUNK), 0)
    onehot = (classes == idx).astype(jnp.float32)
    f = fea_ref[0]
    acc_ref[...] += jax.lax.dot_general(
        onehot, f, (((1,), (1,)), ((), ())),
        preferred_element_type=jnp.float32)
    cnt_ref[:, 0:1] += jnp.sum(onehot, axis=1, keepdims=True)

    @pl.when((b == nb - 1) & (j == nch - 1))
    def _fini():
        cnt = cnt_ref[:, 0:1]
        keys = acc_ref[...] / jnp.maximum(cnt, 1.0)
        norm = jnp.sqrt(jnp.sum(keys * keys, axis=1, keepdims=True))
        out_ref[...] = keys / jnp.maximum(norm, 1e-12)


def _qcopy_body(keys_ref, q_ref, out_ref):
    i = pl.program_id(0)
    j = pl.program_id(1)
    data = q_ref[0]

    @pl.when(j != 0)
    def _copy():
        out_ref[0] = data

    @pl.when(j == 0)
    def _copy0():
        kfull = keys_ref[0]
        sel = jax.lax.broadcasted_iota(jnp.int32, (INNER, NCLASS), 1) == i
        kcol = jnp.sum(jnp.where(sel, kfull, 0.0), axis=1, keepdims=True)
        lane = jax.lax.broadcasted_iota(jnp.int32, (INNER, QCHUNK), 1)
        out_ref[0] = jnp.where(lane == 0, kcol, data)


def kernel(fea, res, queue, batch_size):
    bs = fea.shape[0]
    hw = fea.shape[2] * fea.shape[3]
    nch = hw // CHUNK
    fea3 = fea.reshape(bs, INNER, hw)
    res3 = res.reshape(bs, NCLASS, hw)

    keys8 = pl.pallas_call(
        lambda rr, fr, orr, ar, cr: _keys_body(rr, fr, orr, ar, cr,
                                               nb=bs, nch=nch),
        grid=(bs, nch),
        in_specs=[
            pl.BlockSpec((1, NCLASS, CHUNK), lambda b, j: (b, 0, j)),
            pl.BlockSpec((1, INNER, CHUNK), lambda b, j: (b, 0, j)),
        ],
        out_specs=pl.BlockSpec((8, INNER), lambda b, j: (0, 0)),
        out_shape=jax.ShapeDtypeStruct((8, INNER), jnp.float32),
        scratch_shapes=[
            pltpu.VMEM((8, INNER), jnp.float32),
            pltpu.VMEM((8, 128), jnp.float32),
        ],
        compiler_params=pltpu.CompilerParams(
            dimension_semantics=("arbitrary", "arbitrary")),
    )(res3, fea3)

    keys = keys8[:NCLASS]
    keys_t3 = keys.T.reshape(1, INNER, NCLASS)

    nq = (QUEUE_LEN + QCHUNK - 1) // QCHUNK
    new_queue = pl.pallas_call(
        _qcopy_body,
        grid=(NCLASS, nq),
        in_specs=[
            pl.BlockSpec((1, INNER, NCLASS), lambda i, j: (0, 0, 0)),
            pl.BlockSpec((1, INNER, QCHUNK), lambda i, j: (i, 0, j)),
        ],
        out_specs=pl.BlockSpec((1, INNER, QCHUNK), lambda i, j: (i, 0, j)),
        out_shape=jax.ShapeDtypeStruct((NCLASS, INNER, QUEUE_LEN), jnp.float32),
        compiler_params=pltpu.CompilerParams(
            dimension_semantics=("arbitrary", "arbitrary")),
    )(keys_t3, queue)

    vals = jnp.arange(NCLASS, dtype=jnp.int64)
    return (keys, vals, new_queue)
